# baseline (speedup 1.0000x reference)
import jax
import jax.numpy as jnp
from jax.experimental import pallas as pl
from jax.experimental.pallas import tpu as pltpu

IN_DIM = 256
OUT_DIM = 512
MEM_LEN = 65536
CH = 2048
NCH = MEM_LEN // CH
NBUF = 6


def _dist_body(x_ref, mean_ref, std_ref, wt_ref, b_ref, mem_ref, out_ref,
               minacc, e_scr, bufs, sems):
    i = pl.program_id(0)

    def start(c, b):
        pltpu.make_async_copy(
            mem_ref.at[pl.ds(c * CH, CH)], bufs.at[b], sems.at[b]
        ).start()

    @pl.when(i == 0)
    def _init():
        for b in range(NBUF):
            start(b, b)
        x = x_ref[...]
        mean = mean_ref[...]
        std = std_ref[...]
        new = (x - mean) / std
        new = jnp.where(std == 0, jnp.zeros_like(new), new)
        acc = jnp.dot(new, wt_ref[...], preferred_element_type=jnp.float32)
        e_scr[...] = jnp.tanh(acc + b_ref[...])
        minacc[0] = jnp.float32(jnp.inf)

    slot = i % NBUF
    pltpu.make_async_copy(
        mem_ref.at[pl.ds(0, CH)], bufs.at[slot], sems.at[slot]
    ).wait()

    e = e_scr[...]
    blk = bufs[slot]
    dists = jnp.sum(jnp.abs(blk - e), axis=1)
    minacc[0] = jnp.minimum(minacc[0], jnp.min(dists))

    @pl.when(i + NBUF < NCH)
    def _next():
        start(i + NBUF, slot)

    @pl.when(i == pl.num_programs(0) - 1)
    def _fin():
        out_ref[0] = minacc[0]


def kernel(x, memory, mean, std, W_enc, b_enc):
    xf = x.reshape(1, IN_DIM)
    mean2 = mean.reshape(1, IN_DIM)
    std2 = std.reshape(1, IN_DIM)
    wt = W_enc.T
    b2 = b_enc.reshape(1, OUT_DIM)

    out = pl.pallas_call(
        _dist_body,
        grid=(NCH,),
        in_specs=[
            pl.BlockSpec((1, IN_DIM), lambda i: (0, 0)),
            pl.BlockSpec((1, IN_DIM), lambda i: (0, 0)),
            pl.BlockSpec((1, IN_DIM), lambda i: (0, 0)),
            pl.BlockSpec((IN_DIM, OUT_DIM), lambda i: (0, 0)),
            pl.BlockSpec((1, OUT_DIM), lambda i: (0, 0)),
            pl.BlockSpec(memory_space=pl.ANY),
        ],
        out_specs=pl.BlockSpec(memory_space=pltpu.SMEM),
        out_shape=jax.ShapeDtypeStruct((1,), jnp.float32),
        scratch_shapes=[
            pltpu.SMEM((1,), jnp.float32),
            pltpu.VMEM((1, OUT_DIM), jnp.float32),
            pltpu.VMEM((NBUF, CH, OUT_DIM), jnp.float32),
            pltpu.SemaphoreType.DMA((NBUF,)),
        ],
    )(xf, mean2, std2, wt, b2, memory)
    return out[0]


# speedup vs baseline: 1.0439x; 1.0439x over previous
import jax
import jax.numpy as jnp
from jax.experimental import pallas as pl
from jax.experimental.pallas import tpu as pltpu

IN_DIM = 256
OUT_DIM = 512
MEM_LEN = 65536
BLOCK = 4096


def _dist_body(x_ref, mean_ref, std_ref, wt_ref, b_ref, mem_ref, out_ref,
               minacc, e_scr):
    i = pl.program_id(0)

    @pl.when(i == 0)
    def _init():
        x = x_ref[...]
        mean = mean_ref[...]
        std = std_ref[...]
        new = (x - mean) / std
        new = jnp.where(std == 0, jnp.zeros_like(new), new)
        acc = jnp.dot(new, wt_ref[...], preferred_element_type=jnp.float32)
        e_scr[...] = jnp.tanh(acc + b_ref[...])
        minacc[0] = jnp.float32(jnp.inf)

    e = e_scr[...]
    blk = mem_ref[...]
    dists = jnp.sum(jnp.abs(blk - e), axis=1)
    minacc[0] = jnp.minimum(minacc[0], jnp.min(dists))

    @pl.when(i == pl.num_programs(0) - 1)
    def _fin():
        out_ref[0] = minacc[0]


def kernel(x, memory, mean, std, W_enc, b_enc):
    xf = x.reshape(1, IN_DIM)
    mean2 = mean.reshape(1, IN_DIM)
    std2 = std.reshape(1, IN_DIM)
    wt = W_enc.T
    b2 = b_enc.reshape(1, OUT_DIM)

    grid = MEM_LEN // BLOCK
    out = pl.pallas_call(
        _dist_body,
        grid=(grid,),
        in_specs=[
            pl.BlockSpec((1, IN_DIM), lambda i: (0, 0)),
            pl.BlockSpec((1, IN_DIM), lambda i: (0, 0)),
            pl.BlockSpec((1, IN_DIM), lambda i: (0, 0)),
            pl.BlockSpec((IN_DIM, OUT_DIM), lambda i: (0, 0)),
            pl.BlockSpec((1, OUT_DIM), lambda i: (0, 0)),
            pl.BlockSpec((BLOCK, OUT_DIM), lambda i: (i, 0)),
        ],
        out_specs=pl.BlockSpec(memory_space=pltpu.SMEM),
        out_shape=jax.ShapeDtypeStruct((1,), jnp.float32),
        scratch_shapes=[
            pltpu.SMEM((1,), jnp.float32),
            pltpu.VMEM((1, OUT_DIM), jnp.float32),
        ],
    )(xf, mean2, std2, wt, b2, memory)
    return out[0]
